# baseline (speedup 1.0000x reference)
import jax
import jax.numpy as jnp
from jax.experimental import pallas as pl
from jax.experimental.pallas import tpu as pltpu

E = 16
D = 2048
F = 1024
M = 32
K = 2

NC = 2
EC = E // NC
TF = 512
TD = 1024
T1 = F // TF
T2 = D // TD


def _moe_body(ids_ref, wts_ref, x_ref, gu_ref, dn_ref, out_ref, act_ref):
    ec = pl.program_id(1)
    t = pl.program_id(2)

    @pl.when((ec == 0) & (t == 0))
    def _():
        out_ref[...] = jnp.zeros_like(out_ref)

    @pl.when(t < T1)
    def _():
        act_ref[pl.ds(0, 8), :] = (gu_ref[0, 0, pl.ds(0, 8), pl.ds(0, M)]
                                   + gu_ref[0, 1, pl.ds(0, 8), pl.ds(0, M)])

    @pl.when(t >= T1)
    def _():
        out_ref[0, pl.ds(0, 8), :] = (out_ref[0, pl.ds(0, 8), :]
                                      + dn_ref[0, pl.ds(0, 8), pl.ds(0, M)]
                                      + act_ref[pl.ds(0, 8), :])


def kernel(x, topk_weights, topk_ids, gate_up_proj, down_proj):
    xt = x.T
    ids_t = jnp.pad(topk_ids.T.astype(jnp.int32), ((0, 8 - K), (0, 0)),
                    constant_values=E)
    wts_t = jnp.pad(topk_weights.T, ((0, 8 - K), (0, 0)))
    gu = gate_up_proj.reshape(E, 2, F, D)

    grid = (NC, EC, T1 + T2)
    out_t = pl.pallas_call(
        _moe_body,
        grid=grid,
        in_specs=[
            pl.BlockSpec((8, M), lambda c, e, t: (0, 0)),
            pl.BlockSpec((8, M), lambda c, e, t: (0, 0)),
            pl.BlockSpec((D, M), lambda c, e, t: (0, 0)),
            pl.BlockSpec((1, 2, TF, D),
                         lambda c, e, t: (c * EC + e, 0,
                                          jnp.minimum(t, T1 - 1), 0)),
            pl.BlockSpec((1, TD, F),
                         lambda c, e, t: (c * EC + e,
                                          jnp.where(t < T1, 0, t - T1), 0)),
        ],
        out_specs=pl.BlockSpec((1, D, M), lambda c, e, t: (c, 0, 0)),
        out_shape=jax.ShapeDtypeStruct((NC, D, M), jnp.float32),
        scratch_shapes=[pltpu.VMEM((F, M), jnp.float32)],
        compiler_params=pltpu.CompilerParams(
            dimension_semantics=("parallel", "arbitrary", "arbitrary"),
        ),
    )(ids_t, wts_t, xt, gu, down_proj)
    return (out_t[0] + out_t[1]).T


# speedup vs baseline: 1.0953x; 1.0953x over previous
import jax
import jax.numpy as jnp
from jax.experimental import pallas as pl
from jax.experimental.pallas import tpu as pltpu

E = 16
D = 2048
F = 1024
M = 32
K = 2

TF = 512
T1 = F // TF


def _moe_body(ids_ref, wts_ref, x_ref, gu_ref, dn_ref, out_ref):
    e = pl.program_id(0)
    t = pl.program_id(1)

    g = gu_ref[0, 0]
    u = gu_ref[0, 1]
    xt = x_ref[...]
    hg = jax.lax.dot_general(g, xt, (((1,), (0,)), ((), ())),
                             preferred_element_type=jnp.float32)
    hu = jax.lax.dot_general(u, xt, (((1,), (0,)), ((), ())),
                             preferred_element_type=jnp.float32)
    act = hg / (1.0 + jnp.exp(-hg)) * hu
    dn = dn_ref[0]
    ot = jax.lax.dot_general(dn, act, (((1,), (0,)), ((), ())),
                             preferred_element_type=jnp.float32)
    we = jnp.sum(jnp.where(ids_ref[...] == e, wts_ref[...], 0.0),
                 axis=0, keepdims=True)
    contrib = ot * we

    @pl.when((e == 0) & (t == 0))
    def _():
        out_ref[...] = contrib

    @pl.when((e > 0) | (t > 0))
    def _():
        out_ref[...] = out_ref[...] + contrib


def kernel(x, topk_weights, topk_ids, gate_up_proj, down_proj):
    xt = x.T
    ids_t = jnp.pad(topk_ids.T.astype(jnp.int32), ((0, 8 - K), (0, 0)),
                    constant_values=E)
    wts_t = jnp.pad(topk_weights.T, ((0, 8 - K), (0, 0)))
    gu = gate_up_proj.reshape(E, 2, F, D)

    grid = (E, T1)
    out_t = pl.pallas_call(
        _moe_body,
        grid=grid,
        in_specs=[
            pl.BlockSpec((8, M), lambda e, t: (0, 0)),
            pl.BlockSpec((8, M), lambda e, t: (0, 0)),
            pl.BlockSpec((D, M), lambda e, t: (0, 0)),
            pl.BlockSpec((1, 2, TF, D), lambda e, t: (e, 0, t, 0)),
            pl.BlockSpec((1, D, TF), lambda e, t: (e, 0, t)),
        ],
        out_specs=pl.BlockSpec((D, M), lambda e, t: (0, 0)),
        out_shape=jax.ShapeDtypeStruct((D, M), jnp.float32),
        compiler_params=pltpu.CompilerParams(
            dimension_semantics=("arbitrary", "arbitrary"),
        ),
    )(ids_t, wts_t, xt, gu, down_proj)
    return out_t.T
